# Initial kernel scaffold; baseline (speedup 1.0000x reference)
import jax
import jax.numpy as jnp
from jax.experimental import pallas as pl


def kernel(Y, X, y_idx, x_idx, f):
    raise NotImplementedError("write your pallas kernel here")


# speedup vs baseline: 2.3492x; 2.3492x over previous
import jax
import jax.numpy as jnp
from jax import lax
from jax.experimental import pallas as pl
from jax.experimental.pallas import tpu as pltpu

_B = 8
_G = 2048
_NS = 64
_R = 512


def _body(y_ref, x_ref, yi_ref, xi_ref, f_ref, o_ref):
    j = pl.program_id(1)
    o_ref[...] = y_ref[...]
    r0 = j * _R
    fval = f_ref[0, 0]
    col = lax.broadcasted_iota(jnp.int32, (1, _G), 1)

    def step(s, carry):
        y = yi_ref[s]
        x = xi_ref[s]
        row = y - r0

        @pl.when((row >= 0) & (row < _R))
        def _():
            v = fval * x_ref[0, 0, s]
            o_ref[0, pl.ds(row, 1), :] += jnp.where(col == x, v, 0.0)

        return carry

    lax.fori_loop(0, _NS, step, 0)


def kernel(Y, X, y_idx, x_idx, f):
    f_arr = jnp.asarray(f, jnp.float32).reshape(1, 1)
    grid = (_B, _G // _R)
    return pl.pallas_call(
        _body,
        grid=grid,
        in_specs=[
            pl.BlockSpec((1, _R, _G), lambda b, j: (b, j, 0)),
            pl.BlockSpec((1, 1, _NS), lambda b, j: (b, 0, 0), memory_space=pltpu.SMEM),
            pl.BlockSpec((_NS,), lambda b, j: (0,), memory_space=pltpu.SMEM),
            pl.BlockSpec((_NS,), lambda b, j: (0,), memory_space=pltpu.SMEM),
            pl.BlockSpec((1, 1), lambda b, j: (0, 0), memory_space=pltpu.SMEM),
        ],
        out_specs=pl.BlockSpec((1, _R, _G), lambda b, j: (b, j, 0)),
        out_shape=jax.ShapeDtypeStruct((_B, _G, _G), jnp.float32),
        compiler_params=pltpu.CompilerParams(
            dimension_semantics=("arbitrary", "arbitrary"),
        ),
    )(Y, X.reshape(_B, 1, _NS), y_idx, x_idx, f_arr)
